# baseline (speedup 1.0000x reference)
import functools

import jax
import jax.numpy as jnp
from jax import lax
from jax.experimental import pallas as pl
from jax.experimental.pallas import tpu as pltpu
from jax.experimental.pallas import tpu_sc as plsc

VOCAB_SIZE = 1000000
HALF = VOCAB_SIZE // 2
D = 64
B = 16384
CHUNK = 128


def _make_gather():
    info = plsc.get_sparse_core_info()
    nw = info.num_cores * info.num_subcores
    b_per_w = B // nw
    n_chunks = b_per_w // CHUNK
    mesh = plsc.VectorSubcoreMesh(core_axis_name="c", subcore_axis_name="s")

    @functools.partial(
        pl.kernel,
        mesh=mesh,
        out_type=jax.ShapeDtypeStruct((B, D), jnp.float32),
        scratch_types=[
            pltpu.VMEM((b_per_w,), jnp.int32),
            pltpu.VMEM((b_per_w,), jnp.int32),
            pltpu.VMEM((b_per_w,), jnp.int32),
            pltpu.VMEM((b_per_w, D), jnp.float32),
            pltpu.VMEM((b_per_w, D), jnp.float32),
            pltpu.VMEM((b_per_w, D), jnp.float32),
            pltpu.SemaphoreType.DMA,
        ],
        compiler_params=pltpu.CompilerParams(use_tc_tiling_on_sc=False),
    )
    def k(tab_a, tab_b, idx_hbm, out_hbm, idx_v, ia_v, ib_v, ra_v, rb_v,
          out_v, sem):
        wid = lax.axis_index("s") * info.num_cores + lax.axis_index("c")
        base = wid * b_per_w
        pltpu.sync_copy(idx_hbm.at[pl.ds(base, b_per_w)], idx_v)

        def split(g, _):
            sl = pl.ds(g * 16, 16)
            v = idx_v[sl]
            ia_v[sl] = jnp.minimum(v, HALF - 1)
            ib_v[sl] = jnp.maximum(v - HALF, 0)
            return 0

        lax.fori_loop(0, b_per_w // 16, split, 0)

        for c in range(n_chunks):
            sl = pl.ds(c * CHUNK, CHUNK)
            pltpu.async_copy(tab_a.at[ia_v.at[sl]], ra_v.at[sl], sem)
            pltpu.async_copy(tab_b.at[ib_v.at[sl]], rb_v.at[sl], sem)
        pltpu.make_async_copy(tab_a.at[pl.ds(0, b_per_w)], ra_v, sem).wait()
        pltpu.make_async_copy(tab_b.at[pl.ds(0, b_per_w)], rb_v, sem).wait()

        def pick(g, _):
            gbase = g * 16
            v = idx_v[pl.ds(gbase, 16)]
            for j in range(16):
                i = gbase + j
                in_a = v[j] < HALF
                for col in range(D // 16):
                    sl = pl.ds(col * 16, 16)
                    out_v.at[i][sl] = jnp.where(
                        in_a, ra_v.at[i][sl], rb_v.at[i][sl]
                    )
            return 0

        lax.fori_loop(0, b_per_w // 16, pick, 0)
        pltpu.sync_copy(out_v, out_hbm.at[pl.ds(base, b_per_w)])

    return k


_gather = _make_gather()


def kernel(x, table):
    xi = x.astype(jnp.int32)
    return _gather(table[:HALF], table[HALF:], xi)
